# Initial kernel scaffold; baseline (speedup 1.0000x reference)
import jax
import jax.numpy as jnp
from jax.experimental import pallas as pl


def kernel(cls1, shape1, offset1, cls2, shape2, offset2):
    raise NotImplementedError("write your pallas kernel here")


# speedup vs baseline: 4.4513x; 4.4513x over previous
import jax
import jax.numpy as jnp
from jax import lax
from jax.experimental import pallas as pl
from jax.experimental.pallas import tpu as pltpu

TOPK = 60
THRESHOLD = 0.15
NMS_THRESHOLD = 0.05
NMS_TOPK = 20

N1_ROWS = 864
N2_ROWS = 108
BIG = 2 ** 30
NEG = -1e30


def _extract(x_scr, n_rows, base, vals_l, idxs_l, idxs_s, lanes_s):
    shape = (4, n_rows, 128)
    flat = (lax.broadcasted_iota(jnp.int32, shape, 1) * 128
            + lax.broadcasted_iota(jnp.int32, shape, 2))
    subl = lax.broadcasted_iota(jnp.int32, (4, 128, 1), 1)
    lane = lax.broadcasted_iota(jnp.int32, (4, 1, 128), 2)

    def step(r, carry):
        vals_l, idxs_l, idxs_s, lanes_s = carry
        x = x_scr[...]
        m = jnp.max(jnp.max(x, axis=2, keepdims=True), axis=1, keepdims=True)
        cand = jnp.where(x == m, flat, BIG)
        idx = jnp.min(jnp.min(cand, axis=2, keepdims=True), axis=1,
                      keepdims=True)
        x_scr[...] = jnp.where(flat == idx, -1.0, x)
        pos = base + r
        tgt_l = lane == pos
        tgt_s = subl == pos
        vals_l = jnp.where(tgt_l, m, vals_l)
        idxs_l = jnp.where(tgt_l, idx, idxs_l)
        idxs_s = jnp.where(tgt_s, idx, idxs_s)
        lanes_s = jnp.where(tgt_s, idx % 128, lanes_s)
        return vals_l, idxs_l, idxs_s, lanes_s

    return lax.fori_loop(0, TOPK, step,
                         (vals_l, idxs_l, idxs_s, lanes_s))


def _body(s1_ref, sh1_ref, of1_ref, s2_ref, sh2_ref, of2_ref, out_ref,
          x1_scr, x2_scr):
    x1_scr[...] = 1.0 / (1.0 + jnp.exp(-s1_ref[...]))
    x2_scr[...] = 1.0 / (1.0 + jnp.exp(-s2_ref[...]))

    vals_l = jnp.full((4, 1, 128), -1.0, jnp.float32)
    idxs_l = jnp.zeros((4, 1, 128), jnp.int32)
    idxs_s = jnp.zeros((4, 128, 1), jnp.int32)
    lanes_s = jnp.zeros((4, 128, 1), jnp.int32)
    vals_l, idxs_l, idxs_s, lanes_s = _extract(
        x1_scr, N1_ROWS, 0, vals_l, idxs_l, idxs_s, lanes_s)
    vals_l, idxs_l, idxs_s, lanes_s = _extract(
        x2_scr, N2_ROWS, TOPK, vals_l, idxs_l, idxs_s, lanes_s)

    out_ref[...] = jnp.full((4, 120, 8), -1.0, jnp.float32)

    subl_c = lax.broadcasted_iota(jnp.int32, (128, 1), 0)
    lane_c = lax.broadcasted_iota(jnp.int32, (1, 128), 1)
    eye = (lax.broadcasted_iota(jnp.int32, (128, 128), 0)
           == lax.broadcasted_iota(jnp.int32, (128, 128), 1))
    row1_iota = lax.broadcasted_iota(jnp.int32, (128, N1_ROWS), 1)
    row2_iota = lax.broadcasted_iota(jnp.int32, (128, N2_ROWS), 1)
    lane8 = lax.broadcasted_iota(jnp.int32, (1, 8), 1)

    for b in range(4):
        i_s = idxs_s[b]
        rows = i_s // 128
        lvl2_s = subl_c >= TOPK
        r1 = jnp.where((row1_iota == rows) & jnp.logical_not(lvl2_s),
                       1.0, 0.0)
        r2 = jnp.where((row2_iota == rows) & lvl2_s, 1.0, 0.0)
        lmask = jnp.where(lane_c == lanes_s[b], 1.0, 0.0)

        comps_l = []
        for (a1, a2) in ((sh1_ref, sh2_ref), (of1_ref, of2_ref)):
            for d in range(3):
                g = (jnp.dot(r1, a1[b, d], preferred_element_type=jnp.float32)
                     + jnp.dot(r2, a2[b, d],
                               preferred_element_type=jnp.float32))
                v_s = jnp.sum(g * lmask, axis=1, keepdims=True)
                v_l = jnp.sum(jnp.where(eye, v_s, 0.0), axis=0,
                              keepdims=True)
                comps_l.append(v_l)
        shz, shy, shx, ofz, ofy, ofx = comps_l

        i_l = idxs_l[b]
        lvl2 = lane_c >= TOPK
        stride = jnp.where(lvl2, 4.0, 2.0)
        hw = jnp.where(lvl2, 24 * 24, 48 * 48)
        w = jnp.where(lvl2, 24, 48)
        z = i_l // hw
        rm = i_l - z * hw
        y = rm // w
        x = rm - y * w
        cz = (z.astype(jnp.float32) + ofz) * stride
        cy = (y.astype(jnp.float32) + ofy) * stride
        cx = (x.astype(jnp.float32) + ofx) * stride
        sz = 2.0 * shz * stride
        sy = 2.0 * shy * stride
        sx = 2.0 * shx * stride

        score = vals_l[b]
        vol2 = sz * sy * sx
        lo2z, hi2z = cz - sz * 0.5, cz + sz * 0.5
        lo2y, hi2y = cy - sy * 0.5, cy + sy * 0.5
        lo2x, hi2x = cx - sx * 0.5, cx + sx * 0.5
        cur0 = jnp.where(score > THRESHOLD, score, NEG)

        def nms_step(r, masked):
            m = jnp.max(masked, axis=1, keepdims=True)
            sel_i = jnp.min(jnp.where(masked == m, lane_c, BIG),
                            axis=1, keepdims=True)
            sel = lane_c == sel_i
            ok = m > NEG

            def pick(a):
                return jnp.sum(jnp.where(sel, a, 0.0), axis=1,
                               keepdims=True)
            scs = pick(score)
            czs, cys, cxs = pick(cz), pick(cy), pick(cx)
            szs, sys_, sxs = pick(sz), pick(sy), pick(sx)

            iz = jnp.maximum(jnp.minimum(czs + szs * 0.5, hi2z)
                             - jnp.maximum(czs - szs * 0.5, lo2z), 0.0)
            iy = jnp.maximum(jnp.minimum(cys + sys_ * 0.5, hi2y)
                             - jnp.maximum(cys - sys_ * 0.5, lo2y), 0.0)
            ix = jnp.maximum(jnp.minimum(cxs + sxs * 0.5, hi2x)
                             - jnp.maximum(cxs - sxs * 0.5, lo2x), 0.0)
            inter = iz * iy * ix
            vol1 = szs * sys_ * sxs
            iou = inter / jnp.maximum(vol1 + vol2 - inter, 1e-8)
            keep = (iou <= NMS_THRESHOLD) & jnp.logical_not(sel)
            masked = jnp.where(keep, masked, NEG)

            row = jnp.where(lane8 == 0, 1.0,
                  jnp.where(lane8 == 1, scs,
                  jnp.where(lane8 == 2, czs,
                  jnp.where(lane8 == 3, cys,
                  jnp.where(lane8 == 4, cxs,
                  jnp.where(lane8 == 5, szs,
                  jnp.where(lane8 == 6, sys_, sxs)))))))
            row = jnp.where(ok, row, -1.0)
            out_ref[b, pl.ds(r, 1), :] = row
            return masked

        lax.fori_loop(0, NMS_TOPK, nms_step, cur0)


def kernel(cls1, shape1, offset1, cls2, shape2, offset2):
    s1 = cls1.reshape(4, N1_ROWS, 128)
    sh1 = shape1.reshape(4, 3, N1_ROWS, 128)
    of1 = offset1.reshape(4, 3, N1_ROWS, 128)
    s2 = cls2.reshape(4, N2_ROWS, 128)
    sh2 = shape2.reshape(4, 3, N2_ROWS, 128)
    of2 = offset2.reshape(4, 3, N2_ROWS, 128)
    return pl.pallas_call(
        _body,
        out_shape=jax.ShapeDtypeStruct((4, 120, 8), jnp.float32),
        scratch_shapes=[
            pltpu.VMEM((4, N1_ROWS, 128), jnp.float32),
            pltpu.VMEM((4, N2_ROWS, 128), jnp.float32),
        ],
    )(s1, sh1, of1, s2, sh2, of2)
